# Initial kernel scaffold; baseline (speedup 1.0000x reference)
import jax
import jax.numpy as jnp
from jax.experimental import pallas as pl


def kernel(x, nodes, fact, fact_dim, params, bias):
    raise NotImplementedError("write your pallas kernel here")


# speedup vs baseline: 1.9959x; 1.9959x over previous
import jax
import jax.numpy as jnp
from jax import lax
from jax.experimental import pallas as pl
from jax.experimental.pallas import tpu as pltpu

_MAX_ATOMS = 13
_T = 512


def _grouped_matmul_body(lohi_ref, ids_ref, rn_ref, w_ref, out_ref):
    out_ref[...] = jnp.zeros_like(out_ref)
    lo = lohi_ref[0, 0, 0]
    hi = lohi_ref[0, 0, 1]

    def body(p, carry):
        m = (ids_ref[0] == p).astype(jnp.float32)
        w = w_ref[p]
        out_ref[...] += jnp.dot(rn_ref[...] * m, w,
                                preferred_element_type=jnp.float32)
        return carry

    lax.fori_loop(lo, hi + 1, body, 0)
    out_ref[...] = jnp.maximum(out_ref[...], 0.0)


def kernel(x, nodes, fact, fact_dim, params, bias):
    F = fact.shape[0]
    N, L = nodes.shape
    P, _, R = params.shape
    fact = fact.astype(jnp.int32)

    ids = (x[fact[:, 0], 1].astype(jnp.int32) * _MAX_ATOMS
           + x[fact[:, 0], 2].astype(jnp.int32))
    order = jnp.argsort(ids)
    ids_s = ids[order]

    rows = 2 * F
    rtot = pl.cdiv(rows, _T) * _T
    pad = rtot - rows
    tiles = rtot // _T

    idx_rows = fact[order].ravel()
    nodes_aug = jnp.concatenate(
        [nodes, jnp.ones((N, 1), nodes.dtype), jnp.zeros((N, R - L - 1), nodes.dtype)],
        axis=1)
    rn = jnp.pad(nodes_aug[idx_rows], ((0, pad), (0, 0)))

    ids_rep = jnp.repeat(ids_s, 2)
    ids_p = jnp.pad(ids_rep, (0, pad), constant_values=P - 1)
    lohi = jnp.stack([ids_p[::_T], ids_p[_T - 1::_T]],
                     axis=1).astype(jnp.int32).reshape(tiles, 1, 2)
    ids_b = jnp.broadcast_to(ids_p[:, None], (rtot, R)).reshape(tiles, _T, R)

    w_aug = jnp.concatenate(
        [params, bias, jnp.zeros((P, R - L - 1, R), params.dtype)], axis=1)

    out_sorted = pl.pallas_call(
        _grouped_matmul_body,
        grid=(tiles,),
        in_specs=[
            pl.BlockSpec((1, 1, 2), lambda i: (i, 0, 0), memory_space=pltpu.SMEM),
            pl.BlockSpec((1, _T, R), lambda i: (i, 0, 0)),
            pl.BlockSpec((_T, R), lambda i: (i, 0)),
            pl.BlockSpec((P, R, R), lambda i: (0, 0, 0)),
        ],
        out_specs=pl.BlockSpec((_T, R), lambda i: (i, 0)),
        out_shape=jax.ShapeDtypeStruct((rtot, R), jnp.float32),
    )(lohi, ids_b, rn, w_aug)

    inv = jnp.zeros((F,), jnp.int32).at[order].set(jnp.arange(F, dtype=jnp.int32))
    out2 = out_sorted[:rows].reshape(F, 2, R)
    return jnp.transpose(out2[inv], (1, 0, 2))


# speedup vs baseline: 2.3311x; 1.1679x over previous
import jax
import jax.numpy as jnp
from jax import lax
from jax.experimental import pallas as pl
from jax.experimental.pallas import tpu as pltpu

_MAX_ATOMS = 13
_T = 512


def _grouped_matmul_body(lohi_ref, ids_ref, rn_ref, w_ref, b_ref, out_ref):
    out_ref[...] = jnp.zeros_like(out_ref)
    lo = lohi_ref[0, 0, 0]
    hi = lohi_ref[0, 0, 1]

    def body(p, carry):
        m = (ids_ref[0] == p).astype(jnp.float32)
        contrib = jnp.dot(rn_ref[...] * m, w_ref[p],
                          preferred_element_type=jnp.float32)
        out_ref[...] += contrib + m * b_ref[p]
        return carry

    lax.fori_loop(lo, hi + 1, body, 0)
    out_ref[...] = jnp.maximum(out_ref[...], 0.0)


def kernel(x, nodes, fact, fact_dim, params, bias):
    F = fact.shape[0]
    N, L = nodes.shape
    P, _, R = params.shape
    fact = fact.astype(jnp.int32)

    ids = (x[fact[:, 0], 1].astype(jnp.int32) * _MAX_ATOMS
           + x[fact[:, 0], 2].astype(jnp.int32))
    order = jnp.argsort(ids)
    ids_s = ids[order]

    rows = 2 * F
    rtot = pl.cdiv(rows, _T) * _T
    pad = rtot - rows
    tiles = rtot // _T

    idx_rows = fact[order].ravel()
    rn = jnp.pad(nodes[idx_rows], ((0, pad), (0, 0)))

    ids_rep = jnp.repeat(ids_s, 2)
    ids_p = jnp.pad(ids_rep, (0, pad), constant_values=P - 1)
    lohi = jnp.stack([ids_p[::_T], ids_p[_T - 1::_T]],
                     axis=1).astype(jnp.int32).reshape(tiles, 1, 2)
    ids_b = ids_p.reshape(tiles, _T, 1)

    out_sorted = pl.pallas_call(
        _grouped_matmul_body,
        grid=(tiles,),
        in_specs=[
            pl.BlockSpec((1, 1, 2), lambda i: (i, 0, 0), memory_space=pltpu.SMEM),
            pl.BlockSpec((1, _T, 1), lambda i: (i, 0, 0)),
            pl.BlockSpec((_T, L), lambda i: (i, 0)),
            pl.BlockSpec((P, L, R), lambda i: (0, 0, 0)),
            pl.BlockSpec((P, 1, R), lambda i: (0, 0, 0)),
        ],
        out_specs=pl.BlockSpec((_T, R), lambda i: (i, 0)),
        out_shape=jax.ShapeDtypeStruct((rtot, R), jnp.float32),
    )(lohi, ids_b, rn, params, bias)

    inv = jnp.zeros((F,), jnp.int32).at[order].set(jnp.arange(F, dtype=jnp.int32))
    src_rows = (2 * inv[None, :] + jnp.arange(2, dtype=jnp.int32)[:, None]).ravel()
    return out_sorted[src_rows].reshape(2, F, R)
